# Initial kernel scaffold; baseline (speedup 1.0000x reference)
import jax
import jax.numpy as jnp
from jax.experimental import pallas as pl


def kernel(img, label, We, be, Wd, bd):
    raise NotImplementedError("write your pallas kernel here")


# speedup vs baseline: 4.1375x; 4.1375x over previous
import functools

import jax
import jax.numpy as jnp
from jax.experimental import pallas as pl
from jax.experimental.pallas import tpu as pltpu


def _fused_body(nb, inv_nd, lab_ref, img_ref, we2_ref, be2_ref, wd2_ref,
                bd_ref, out_ref, loss_ref):
    i = pl.program_id(0)
    lab = lab_ref[...]
    x = img_ref[...]
    eh = we2_ref.shape[1]
    e = bd_ref.shape[0]
    h = eh // e

    hid = jnp.dot(x, we2_ref[...], preferred_element_type=jnp.float32)
    hid = hid + be2_ref[...]
    colid = jax.lax.broadcasted_iota(jnp.int32, (1, eh), 1) // h
    hid = jnp.where(lab == colid, hid, 0.0)

    onehot = (lab == jax.lax.broadcasted_iota(jnp.int32, (1, e), 1))
    y = jnp.dot(hid, wd2_ref[...], preferred_element_type=jnp.float32)
    y = y + jnp.dot(onehot.astype(jnp.float32), bd_ref[...],
                    preferred_element_type=jnp.float32)
    out_ref[...] = y

    d = y - x

    @pl.when(i == 0)
    def _():
        loss_ref[...] = jnp.zeros_like(loss_ref)

    loss_ref[...] += jnp.sum(d * d)

    @pl.when(i == nb - 1)
    def _():
        loss_ref[...] *= inv_nd


def kernel(img, label, We, be, Wd, bd):
    n, d = img.shape
    e, _, h = We.shape
    tb = 512
    nb = n // tb

    we2 = We.transpose(1, 0, 2).reshape(d, e * h)
    be2 = be.reshape(1, e * h)
    wd2 = Wd.reshape(e * h, d)
    lab2 = label.astype(jnp.int32).reshape(n, 1)

    out, loss = pl.pallas_call(
        functools.partial(_fused_body, nb, 1.0 / (n * d)),
        grid=(nb,),
        in_specs=[
            pl.BlockSpec((tb, 1), lambda i: (i, 0)),
            pl.BlockSpec((tb, d), lambda i: (i, 0)),
            pl.BlockSpec((d, e * h), lambda i: (0, 0)),
            pl.BlockSpec((1, e * h), lambda i: (0, 0)),
            pl.BlockSpec((e * h, d), lambda i: (0, 0)),
            pl.BlockSpec((e, d), lambda i: (0, 0)),
        ],
        out_specs=[
            pl.BlockSpec((tb, d), lambda i: (i, 0)),
            pl.BlockSpec((1, 1), lambda i: (0, 0)),
        ],
        out_shape=[
            jax.ShapeDtypeStruct((n, d), jnp.float32),
            jax.ShapeDtypeStruct((1, 1), jnp.float32),
        ],
    )(lab2, img, we2, be2, wd2, bd)
    return (loss[0, 0], out)
